# baseline (speedup 1.0000x reference)
import functools

import jax
import jax.numpy as jnp
from jax import lax
from jax.experimental import pallas as pl
from jax.experimental.pallas import tpu as pltpu
from jax.experimental.pallas import tpu_sc as plsc

_BATCH = 4
_SEQ = 4096
_HIDDEN = 1024
_NROWS = _BATCH * _SEQ
_NC = 2
_NS = 16
_NW = _NC * _NS
_PER_W = _NROWS // _NW
_CHUNK = 32
_NCHUNK = _PER_W // _CHUNK
_LANES = 16
_COLS = _HIDDEN // _LANES


def _emb_kernel(ids_hbm, word_hbm, pos_hbm, out_hbm, idx_v, wbuf, pbuf, sem):
    wid = lax.axis_index("s") * _NC + lax.axis_index("c")
    base = wid * _PER_W
    pltpu.sync_copy(ids_hbm.at[pl.ds(base, _PER_W)], idx_v)

    def chunk_body(ci, carry):
        row0 = base + ci * _CHUNK
        pos0 = lax.rem(row0, _SEQ)
        pltpu.async_copy(
            word_hbm.at[idx_v.at[pl.ds(ci * _CHUNK, _CHUNK)]], wbuf, sem
        ).wait()
        pltpu.sync_copy(pos_hbm.at[pl.ds(pos0, _CHUNK)], pbuf)

        def add_row(r, c2):
            def add_col(c, c3):
                sl = pl.ds(c * _LANES, _LANES)
                wbuf[r, sl] = wbuf[r, sl] + pbuf[r, sl]
                return c3
            return lax.fori_loop(0, _COLS, add_col, c2)

        lax.fori_loop(0, _CHUNK, add_row, 0)
        pltpu.sync_copy(wbuf, out_hbm.at[pl.ds(row0, _CHUNK)])
        return carry

    lax.fori_loop(0, _NCHUNK, chunk_body, 0)


@jax.jit
def _run(ids_flat, word_table, pos_table):
    mesh = plsc.VectorSubcoreMesh(core_axis_name="c", subcore_axis_name="s")
    f = functools.partial(
        pl.kernel,
        out_type=jax.ShapeDtypeStruct((_NROWS, _HIDDEN), jnp.float32),
        mesh=mesh,
        scratch_types=[
            pltpu.VMEM((_PER_W,), jnp.int32),
            pltpu.VMEM((_CHUNK, _HIDDEN), jnp.float32),
            pltpu.VMEM((_CHUNK, _HIDDEN), jnp.float32),
            pltpu.SemaphoreType.DMA,
        ],
    )(_emb_kernel)
    return f(ids_flat, word_table, pos_table)


def kernel(input_ids, word_table, pos_table):
    ids_flat = input_ids.reshape(-1).astype(jnp.int32)
    out = _run(ids_flat, word_table, pos_table)
    return out.reshape(_BATCH, _SEQ, _HIDDEN)


# speedup vs baseline: 2.3495x; 2.3495x over previous
import functools

import jax
import jax.numpy as jnp
from jax import lax
from jax.experimental import pallas as pl
from jax.experimental.pallas import tpu as pltpu
from jax.experimental.pallas import tpu_sc as plsc

_BATCH = 4
_SEQ = 4096
_HIDDEN = 1024
_NROWS = _BATCH * _SEQ
_NC = 2
_NS = 16
_NW = _NC * _NS
_POS_W = _SEQ // _NW
_PCHUNK = 8
_NCHUNK = _POS_W // _PCHUNK
_LANES = 16
_COLS = _HIDDEN // _LANES


def _emb_kernel(ids_hbm, word_hbm, pos_hbm, out_hbm, idx_v, wbuf, pbuf, sem):
    wid = lax.axis_index("s") * _NC + lax.axis_index("c")
    p_base = wid * _POS_W
    for b in range(_BATCH):
        pltpu.sync_copy(ids_hbm.at[pl.ds(b * _SEQ + p_base, _POS_W)],
                        idx_v.at[b])

    def chunk_body(ci, carry):
        pos0 = p_base + ci * _PCHUNK
        cps = [
            pltpu.async_copy(
                word_hbm.at[idx_v.at[b, pl.ds(ci * _PCHUNK, _PCHUNK)]],
                wbuf.at[b], sem)
            for b in range(_BATCH)
        ]
        pltpu.sync_copy(pos_hbm.at[pl.ds(pos0, _PCHUNK)], pbuf)
        for cp in cps:
            cp.wait()

        def add_row(p, c2):
            for c in range(_COLS):
                sl = pl.ds(c * _LANES, _LANES)
                pv = pbuf[p, sl]
                for b in range(_BATCH):
                    wbuf[b, p, sl] = wbuf[b, p, sl] + pv
            return c2

        lax.fori_loop(0, _PCHUNK, add_row, 0)
        for b in range(_BATCH):
            pltpu.sync_copy(wbuf.at[b],
                            out_hbm.at[pl.ds(b * _SEQ + pos0, _PCHUNK)])
        return carry

    lax.fori_loop(0, _NCHUNK, chunk_body, 0)


@jax.jit
def _run(ids_flat, word_table, pos_table):
    mesh = plsc.VectorSubcoreMesh(core_axis_name="c", subcore_axis_name="s")
    f = functools.partial(
        pl.kernel,
        out_type=jax.ShapeDtypeStruct((_NROWS, _HIDDEN), jnp.float32),
        mesh=mesh,
        scratch_types=[
            pltpu.VMEM((_BATCH, _POS_W), jnp.int32),
            pltpu.VMEM((_BATCH, _PCHUNK, _HIDDEN), jnp.float32),
            pltpu.VMEM((_PCHUNK, _HIDDEN), jnp.float32),
            pltpu.SemaphoreType.DMA,
        ],
    )(_emb_kernel)
    return f(ids_flat, word_table, pos_table)


def kernel(input_ids, word_table, pos_table):
    ids_flat = input_ids.reshape(-1).astype(jnp.int32)
    out = _run(ids_flat, word_table, pos_table)
    return out.reshape(_BATCH, _SEQ, _HIDDEN)


# speedup vs baseline: 3.3074x; 1.4077x over previous
import functools

import jax
import jax.numpy as jnp
from jax import lax
from jax.experimental import pallas as pl
from jax.experimental.pallas import tpu as pltpu
from jax.experimental.pallas import tpu_sc as plsc

_BATCH = 4
_SEQ = 4096
_HIDDEN = 1024
_NROWS = _BATCH * _SEQ
_NC = 2
_NS = 16
_NW = _NC * _NS
_POS_W = _SEQ // _NW
_PCHUNK = 8
_NCHUNK = _POS_W // _PCHUNK
_NPAIR = _NCHUNK // 2
_LANES = 16
_COLS = _HIDDEN // _LANES


def _emb_kernel(ids_hbm, word_hbm, pos_hbm, out_hbm, idx_v,
                wbuf0, wbuf1, pbuf0, pbuf1, gsem0, gsem1, ssem0, ssem1):
    wid = lax.axis_index("s") * _NC + lax.axis_index("c")
    p_base = wid * _POS_W
    wbufs, pbufs = (wbuf0, wbuf1), (pbuf0, pbuf1)
    gsems, ssems = (gsem0, gsem1), (ssem0, ssem1)

    for b in range(_BATCH):
        pltpu.sync_copy(ids_hbm.at[pl.ds(b * _SEQ + p_base, _POS_W)],
                        idx_v.at[b])

    def in_copies(ci, par):
        pos0 = p_base + ci * _PCHUNK
        cps = [
            pltpu.make_async_copy(
                word_hbm.at[idx_v.at[b, pl.ds(ci * _PCHUNK, _PCHUNK)]],
                wbufs[par].at[b], gsems[par])
            for b in range(_BATCH)
        ]
        cps.append(pltpu.make_async_copy(
            pos_hbm.at[pl.ds(pos0, _PCHUNK)], pbufs[par], gsems[par]))
        return cps

    def out_copies(ci, par):
        pos0 = p_base + ci * _PCHUNK
        return [
            pltpu.make_async_copy(
                wbufs[par].at[b],
                out_hbm.at[pl.ds(b * _SEQ + pos0, _PCHUNK)], ssems[par])
            for b in range(_BATCH)
        ]

    for cp in in_copies(0, 0):
        cp.start()

    def pair_body(ci2, carry):
        for par in range(2):
            ci = ci2 * 2 + par
            if par == 1:
                for cp in out_copies(ci - 1, 0):
                    cp.wait()
            else:
                @pl.when(ci2 > 0)
                def _():
                    for cp in out_copies(ci - 1, 1):
                        cp.wait()
            if par == 0:
                for cp in in_copies(ci + 1, 1):
                    cp.start()
            else:
                @pl.when(ci2 < _NPAIR - 1)
                def _():
                    for cp in in_copies(ci + 1, 0):
                        cp.start()
            for cp in in_copies(ci, par):
                cp.wait()

            wb, pb = wbufs[par], pbufs[par]

            def add_row(p, c2):
                for c in range(_COLS):
                    sl = pl.ds(c * _LANES, _LANES)
                    pv = pb[p, sl]
                    for b in range(_BATCH):
                        wb[b, p, sl] = wb[b, p, sl] + pv
                return c2

            lax.fori_loop(0, _PCHUNK, add_row, 0)

            for cp in out_copies(ci, par):
                cp.start()
        return carry

    lax.fori_loop(0, _NPAIR, pair_body, 0)
    for cp in out_copies(_NCHUNK - 1, 1):
        cp.wait()


@jax.jit
def _run(ids_flat, word_table, pos_table):
    mesh = plsc.VectorSubcoreMesh(core_axis_name="c", subcore_axis_name="s")
    f = functools.partial(
        pl.kernel,
        out_type=jax.ShapeDtypeStruct((_NROWS, _HIDDEN), jnp.float32),
        mesh=mesh,
        scratch_types=[
            pltpu.VMEM((_BATCH, _POS_W), jnp.int32),
            pltpu.VMEM((_BATCH, _PCHUNK, _HIDDEN), jnp.float32),
            pltpu.VMEM((_BATCH, _PCHUNK, _HIDDEN), jnp.float32),
            pltpu.VMEM((_PCHUNK, _HIDDEN), jnp.float32),
            pltpu.VMEM((_PCHUNK, _HIDDEN), jnp.float32),
            pltpu.SemaphoreType.DMA,
            pltpu.SemaphoreType.DMA,
            pltpu.SemaphoreType.DMA,
            pltpu.SemaphoreType.DMA,
        ],
    )(_emb_kernel)
    return f(ids_flat, word_table, pos_table)


def kernel(input_ids, word_table, pos_table):
    ids_flat = input_ids.reshape(-1).astype(jnp.int32)
    out = _run(ids_flat, word_table, pos_table)
    return out.reshape(_BATCH, _SEQ, _HIDDEN)
